# Initial kernel scaffold; baseline (speedup 1.0000x reference)
import functools

import jax
import jax.numpy as jnp
from jax import lax
from jax.experimental import pallas as pl
from jax.experimental.pallas import tpu as pltpu
from jax.experimental.pallas import tpu_sc as plsc

N = 10000
CH = 128
E = 320000
NW = 32
EPW = E // NW
B = 80
K = EPW // B
RPT = N // 16
NPAD = 10240
RPD = NPAD // 16

_mesh = plsc.VectorSubcoreMesh(
    core_axis_name="c", subcore_axis_name="s", num_cores=2, num_subcores=16)

_f32 = jnp.float32


@functools.partial(
    pl.kernel,
    out_type=jax.ShapeDtypeStruct((NPAD,), _f32),
    mesh=_mesh,
    scratch_types=[
        pltpu.VMEM_SHARED((NPAD,), _f32),
        pltpu.VMEM((K, B), jnp.int32),
        pltpu.VMEM((B,), _f32),
        pltpu.VMEM((RPD,), _f32),
    ],
)
def _deg_kernel(dst_hbm, dinv_hbm, hist_sp, idx_v, ones_v, buf_v):
    c = lax.axis_index("c")
    s = lax.axis_index("s")

    @pl.when(c == 0)
    def _():
        for i in range(RPD // 16):
            buf_v[pl.ds(16 * i, 16)] = jnp.zeros((16,), _f32)
        pltpu.sync_copy(buf_v, hist_sp.at[pl.ds(RPD * s, RPD)])
        for i in range(B // 16):
            ones_v[pl.ds(16 * i, 16)] = jnp.ones((16,), _f32)
        plsc.subcore_barrier()

        def one_worker(w):
            pltpu.sync_copy(dst_hbm.at[w], idx_v)

            def body(k, carry):
                pltpu.sync_copy(ones_v, hist_sp.at[idx_v.at[k]], add=True)
                return carry

            lax.fori_loop(0, K, body, 0)

        one_worker(2 * s)
        one_worker(2 * s + 1)
        plsc.subcore_barrier()

        pltpu.sync_copy(hist_sp.at[pl.ds(RPD * s, RPD)], buf_v)
        for i in range(RPD // 16):
            d = buf_v[pl.ds(16 * i, 16)] + 1.0
            bits = plsc.bitcast(d, jnp.int32)
            y = plsc.bitcast(jnp.int32(0x5F3759DF) - (bits >> 1), _f32)
            for _ in range(3):
                y = y * (1.5 - 0.5 * d * y * y)
            buf_v[pl.ds(16 * i, 16)] = y
        pltpu.sync_copy(buf_v, dinv_hbm.at[pl.ds(RPD * s, RPD)])


@functools.partial(
    pl.kernel,
    out_type=jax.ShapeDtypeStruct((2, N, CH), _f32),
    mesh=_mesh,
    scratch_types=[
        pltpu.VMEM_SHARED((N, CH), _f32),
        pltpu.VMEM((K, B), jnp.int32),
        pltpu.VMEM((K, B), jnp.int32),
        pltpu.VMEM((B, CH), _f32),
        pltpu.SemaphoreType.DMA,
    ],
)
def _agg_kernel(z_hbm, src_hbm, dst_hbm, out_hbm,
                acc_sp, src_v, dst_v, rows_v, sem):
    c = lax.axis_index("c")
    s = lax.axis_index("s")
    w = c * 16 + s
    row0 = RPT * s
    pltpu.sync_copy(z_hbm.at[pl.ds(row0, RPT)], acc_sp.at[pl.ds(row0, RPT)])
    pltpu.sync_copy(src_hbm.at[w], src_v)
    pltpu.sync_copy(dst_hbm.at[w], dst_v)
    plsc.subcore_barrier()

    def body(k, carry):
        pltpu.async_copy(z_hbm.at[src_v.at[k]], rows_v, sem).wait()
        pltpu.sync_copy(rows_v, acc_sp.at[dst_v.at[k]], add=True)
        return carry

    lax.fori_loop(0, K, body, 0)
    plsc.subcore_barrier()
    pltpu.sync_copy(acc_sp.at[pl.ds(row0, RPT)],
                    out_hbm.at[c, pl.ds(row0, RPT)])


def _mm_body(x_ref, w_ref, d_ref, z_ref):
    z_ref[...] = jnp.dot(x_ref[...], w_ref[...],
                         preferred_element_type=_f32) * d_ref[...]


def _fin_body(a0_ref, a1_ref, z_ref, d_ref, b_ref, o_ref):
    t = d_ref[...] * (a0_ref[...] + a1_ref[...] - z_ref[...]) + b_ref[...]
    o_ref[...] = jnp.where(t >= 0.0, 2.0 * t, 1.01 * t)


_BM = 400


def _mm(x, W, dinv_col):
    return pl.pallas_call(
        _mm_body,
        grid=(N // _BM,),
        in_specs=[
            pl.BlockSpec((_BM, CH), lambda i: (i, 0)),
            pl.BlockSpec((CH, CH), lambda i: (0, 0)),
            pl.BlockSpec((_BM, 1), lambda i: (i, 0)),
        ],
        out_specs=pl.BlockSpec((_BM, CH), lambda i: (i, 0)),
        out_shape=jax.ShapeDtypeStruct((N, CH), _f32),
    )(x, W, dinv_col)


def _fin(a0, a1, z, dinv_col, b2):
    return pl.pallas_call(
        _fin_body,
        grid=(N // _BM,),
        in_specs=[
            pl.BlockSpec((_BM, CH), lambda i: (i, 0)),
            pl.BlockSpec((_BM, CH), lambda i: (i, 0)),
            pl.BlockSpec((_BM, CH), lambda i: (i, 0)),
            pl.BlockSpec((_BM, 1), lambda i: (i, 0)),
            pl.BlockSpec((1, CH), lambda i: (0, 0)),
        ],
        out_specs=pl.BlockSpec((_BM, CH), lambda i: (i, 0)),
        out_shape=jax.ShapeDtypeStruct((N, CH), _f32),
    )(a0, a1, z, dinv_col, b2)


@jax.jit
def kernel(x, edges, W, b):
    src = edges[0].astype(jnp.int32).reshape(NW, K, B)
    dst = edges[1].astype(jnp.int32).reshape(NW, K, B)
    dinv = _deg_kernel(dst)[:N]
    dinv_col = dinv[:, None]
    z = _mm(x, W, dinv_col)
    acc = _agg_kernel(z, src, dst)
    return _fin(acc[0], acc[1], z, dinv_col, b[None, :])


# speedup vs baseline: 24.3833x; 24.3833x over previous
import functools

import jax
import jax.numpy as jnp
from jax import lax
from jax.experimental import pallas as pl
from jax.experimental.pallas import tpu as pltpu
from jax.experimental.pallas import tpu_sc as plsc

N = 10000
CH = 128
E = 320000
NW = 32
EPW = E // NW
B = 80
K = EPW // B
RPT = 624
NPAD = 10240
RPD = NPAD // 16

_mesh = plsc.VectorSubcoreMesh(
    core_axis_name="c", subcore_axis_name="s", num_cores=2, num_subcores=16)

_f32 = jnp.float32


@functools.partial(
    pl.kernel,
    out_type=jax.ShapeDtypeStruct((NPAD,), _f32),
    mesh=_mesh,
    scratch_types=[
        pltpu.VMEM_SHARED((NPAD,), _f32),
        pltpu.VMEM((K, B), jnp.int32),
        pltpu.VMEM((B,), _f32),
        pltpu.VMEM((RPD,), _f32),
    ],
)
def _deg_kernel(dst_hbm, hist_hbm, hist_sp, idx_v, ones_v, buf_v):
    c = lax.axis_index("c")
    s = lax.axis_index("s")

    @pl.when(c == 0)
    def _():
        for i in range(RPD // 16):
            buf_v[pl.ds(16 * i, 16)] = jnp.zeros((16,), _f32)
        pltpu.sync_copy(buf_v, hist_sp.at[pl.ds(RPD * s, RPD)])
        for i in range(B // 16):
            ones_v[pl.ds(16 * i, 16)] = jnp.ones((16,), _f32)
        plsc.subcore_barrier()

        def one_worker(w):
            pltpu.sync_copy(dst_hbm.at[w], idx_v)

            def body(k, carry):
                pltpu.sync_copy(ones_v, hist_sp.at[idx_v.at[k]], add=True)
                return carry

            lax.fori_loop(0, K, body, 0)

        one_worker(2 * s)
        one_worker(2 * s + 1)
        plsc.subcore_barrier()

        @pl.when(s == 0)
        def _():
            pltpu.sync_copy(hist_sp, hist_hbm)


@functools.partial(
    pl.kernel,
    out_type=jax.ShapeDtypeStruct((2, N, CH), _f32),
    mesh=_mesh,
    scratch_types=[
        pltpu.VMEM_SHARED((N, CH), _f32),
        pltpu.VMEM((K, B), jnp.int32),
        pltpu.VMEM((K, B), jnp.int32),
        pltpu.VMEM((B, CH), _f32),
        pltpu.SemaphoreType.DMA,
    ],
)
def _agg_kernel(z_hbm, src_hbm, dst_hbm, out_hbm,
                acc_sp, src_v, dst_v, rows_v, sem):
    c = lax.axis_index("c")
    s = lax.axis_index("s")
    w = c * 16 + s
    row0 = pl.multiple_of(RPT * s, 8)
    pltpu.sync_copy(z_hbm.at[pl.ds(row0, RPT)], acc_sp.at[pl.ds(row0, RPT)])

    @pl.when(s == 0)
    def _():
        pltpu.sync_copy(z_hbm.at[pl.ds(16 * RPT, N - 16 * RPT)],
                        acc_sp.at[pl.ds(16 * RPT, N - 16 * RPT)])

    pltpu.sync_copy(src_hbm.at[w], src_v)
    pltpu.sync_copy(dst_hbm.at[w], dst_v)
    plsc.subcore_barrier()

    def body(k, carry):
        pltpu.async_copy(z_hbm.at[src_v.at[k]], rows_v, sem).wait()
        pltpu.sync_copy(rows_v, acc_sp.at[dst_v.at[k]], add=True)
        return carry

    lax.fori_loop(0, K, body, 0)
    plsc.subcore_barrier()
    pltpu.sync_copy(acc_sp.at[pl.ds(row0, RPT)],
                    out_hbm.at[c, pl.ds(row0, RPT)])

    @pl.when(s == 0)
    def _():
        pltpu.sync_copy(acc_sp.at[pl.ds(16 * RPT, N - 16 * RPT)],
                        out_hbm.at[c, pl.ds(16 * RPT, N - 16 * RPT)])


def _rsq_body(h_ref, d_ref):
    d_ref[...] = lax.rsqrt(h_ref[...] + 1.0)


def _rsq(hist2d):
    return pl.pallas_call(
        _rsq_body,
        out_shape=jax.ShapeDtypeStruct(hist2d.shape, _f32),
    )(hist2d)


def _mm_body(x_ref, w_ref, d_ref, z_ref):
    z_ref[...] = jnp.dot(x_ref[...], w_ref[...],
                         preferred_element_type=_f32) * d_ref[...]


def _fin_body(a0_ref, a1_ref, z_ref, d_ref, b_ref, o_ref):
    t = d_ref[...] * (a0_ref[...] + a1_ref[...] - z_ref[...]) + b_ref[...]
    o_ref[...] = jnp.where(t >= 0.0, 2.0 * t, 1.01 * t)


_BM = 400


def _mm(x, W, dinv_col):
    return pl.pallas_call(
        _mm_body,
        grid=(N // _BM,),
        in_specs=[
            pl.BlockSpec((_BM, CH), lambda i: (i, 0)),
            pl.BlockSpec((CH, CH), lambda i: (0, 0)),
            pl.BlockSpec((_BM, 1), lambda i: (i, 0)),
        ],
        out_specs=pl.BlockSpec((_BM, CH), lambda i: (i, 0)),
        out_shape=jax.ShapeDtypeStruct((N, CH), _f32),
    )(x, W, dinv_col)


def _fin(a0, a1, z, dinv_col, b2):
    return pl.pallas_call(
        _fin_body,
        grid=(N // _BM,),
        in_specs=[
            pl.BlockSpec((_BM, CH), lambda i: (i, 0)),
            pl.BlockSpec((_BM, CH), lambda i: (i, 0)),
            pl.BlockSpec((_BM, CH), lambda i: (i, 0)),
            pl.BlockSpec((_BM, 1), lambda i: (i, 0)),
            pl.BlockSpec((1, CH), lambda i: (0, 0)),
        ],
        out_specs=pl.BlockSpec((_BM, CH), lambda i: (i, 0)),
        out_shape=jax.ShapeDtypeStruct((N, CH), _f32),
    )(a0, a1, z, dinv_col, b2)


@jax.jit
def kernel(x, edges, W, b):
    src = edges[0].astype(jnp.int32).reshape(NW, K, B)
    dst = edges[1].astype(jnp.int32).reshape(NW, K, B)
    hist = _deg_kernel(dst)
    dinv = _rsq(hist.reshape(NPAD // CH, CH)).reshape(NPAD)[:N]
    dinv_col = dinv[:, None]
    z = _mm(x, W, dinv_col)
    acc = _agg_kernel(z, src, dst)
    return _fin(acc[0], acc[1], z, dinv_col, b[None, :])


# speedup vs baseline: 35.2697x; 1.4465x over previous
import functools

import jax
import jax.numpy as jnp
from jax import lax
from jax.experimental import pallas as pl
from jax.experimental.pallas import tpu as pltpu
from jax.experimental.pallas import tpu_sc as plsc

N = 10000
CH = 128
E = 320000
NW = 32
EPW = E // NW
BA = 125
KA = EPW // BA
BD = 128
KD = 80
EPAD = KD * BD - EPW
RPT = 624
NPAD = 10240
RPD = NPAD // 16

_mesh = plsc.VectorSubcoreMesh(
    core_axis_name="c", subcore_axis_name="s", num_cores=2, num_subcores=16)

_f32 = jnp.float32


@functools.partial(
    pl.kernel,
    out_type=jax.ShapeDtypeStruct((2, NPAD), _f32),
    mesh=_mesh,
    scratch_types=[
        pltpu.VMEM_SHARED((NPAD,), _f32),
        pltpu.VMEM((KD, BD), jnp.int32),
        pltpu.VMEM((BD,), _f32),
        pltpu.VMEM((RPD,), _f32),
    ],
)
def _deg_kernel(dst_hbm, hist_hbm, hist_sp, idx_v, ones_v, buf_v):
    c = lax.axis_index("c")
    s = lax.axis_index("s")
    for i in range(RPD // 16):
        buf_v[pl.ds(16 * i, 16)] = jnp.zeros((16,), _f32)
    pltpu.sync_copy(buf_v, hist_sp.at[pl.ds(RPD * s, RPD)])
    for i in range(BD // 16):
        ones_v[pl.ds(16 * i, 16)] = jnp.ones((16,), _f32)
    pltpu.sync_copy(dst_hbm.at[c * 16 + s], idx_v)
    plsc.subcore_barrier()

    def body(k, carry):
        pltpu.sync_copy(ones_v, hist_sp.at[idx_v.at[k]], add=True)
        return carry

    lax.fori_loop(0, KD, body, 0)
    plsc.subcore_barrier()

    @pl.when(s == 0)
    def _():
        pltpu.sync_copy(hist_sp, hist_hbm.at[c])


@functools.partial(
    pl.kernel,
    out_type=jax.ShapeDtypeStruct((2, N, CH), _f32),
    mesh=_mesh,
    scratch_types=[
        pltpu.VMEM_SHARED((N, CH), _f32),
        pltpu.VMEM((2, BA), jnp.int32),
        pltpu.VMEM((2, BA), jnp.int32),
        pltpu.VMEM((BA, CH), _f32),
        pltpu.VMEM((BA, CH), _f32),
        pltpu.SemaphoreType.DMA,
        pltpu.SemaphoreType.DMA,
        pltpu.SemaphoreType.DMA,
        pltpu.SemaphoreType.DMA,
    ],
)
def _agg_kernel(z_hbm, idx_hbm, out_hbm,
                acc_sp, ib0, ib1, rows_v0, rows_v1,
                semi0, semi1, semr0, semr1):
    c = lax.axis_index("c")
    s = lax.axis_index("s")
    w = c * 16 + s
    row0 = pl.multiple_of(RPT * s, 8)
    pltpu.sync_copy(z_hbm.at[pl.ds(row0, RPT)], acc_sp.at[pl.ds(row0, RPT)])

    @pl.when(s == 0)
    def _():
        pltpu.sync_copy(z_hbm.at[pl.ds(16 * RPT, N - 16 * RPT)],
                        acc_sp.at[pl.ds(16 * RPT, N - 16 * RPT)])

    def load_idx(k, ib, semi):
        pltpu.async_copy(idx_hbm.at[w, k], ib, semi)

    def wait_idx(ib, semi):
        pltpu.make_async_copy(idx_hbm.at[w, 0], ib, semi).wait()

    def gather(ib, buf, semr):
        pltpu.async_copy(z_hbm.at[ib.at[0]], buf, semr)

    def wait_rows(buf, semr):
        pltpu.make_async_copy(z_hbm.at[ib0.at[0]], buf, semr).wait()

    def scatter(ib, buf):
        pltpu.sync_copy(buf, acc_sp.at[ib.at[1]], add=True)

    pltpu.sync_copy(idx_hbm.at[w, 0], ib0)
    plsc.subcore_barrier()
    gather(ib0, rows_v0, semr0)
    load_idx(1, ib1, semi1)

    def body(j, carry):
        k = 2 * j
        wait_idx(ib1, semi1)
        gather(ib1, rows_v1, semr1)
        wait_rows(rows_v0, semr0)
        scatter(ib0, rows_v0)
        load_idx(k + 2, ib0, semi0)
        wait_idx(ib0, semi0)
        gather(ib0, rows_v0, semr0)
        wait_rows(rows_v1, semr1)
        scatter(ib1, rows_v1)
        load_idx(k + 3, ib1, semi1)
        return carry

    lax.fori_loop(0, (KA - 2) // 2, body, 0)
    wait_idx(ib1, semi1)
    gather(ib1, rows_v1, semr1)
    wait_rows(rows_v0, semr0)
    scatter(ib0, rows_v0)
    wait_rows(rows_v1, semr1)
    scatter(ib1, rows_v1)
    plsc.subcore_barrier()
    pltpu.sync_copy(acc_sp.at[pl.ds(row0, RPT)],
                    out_hbm.at[c, pl.ds(row0, RPT)])

    @pl.when(s == 0)
    def _():
        pltpu.sync_copy(acc_sp.at[pl.ds(16 * RPT, N - 16 * RPT)],
                        out_hbm.at[c, pl.ds(16 * RPT, N - 16 * RPT)])


def _rsq_body(h0_ref, h1_ref, d_ref):
    d_ref[...] = lax.rsqrt(h0_ref[...] + h1_ref[...] + 1.0)


def _rsq(h0, h1):
    return pl.pallas_call(
        _rsq_body,
        out_shape=jax.ShapeDtypeStruct(h0.shape, _f32),
    )(h0, h1)


def _mm_body(x_ref, w_ref, d_ref, z_ref):
    z_ref[...] = jnp.dot(x_ref[...], w_ref[...],
                         preferred_element_type=_f32) * d_ref[...]


def _fin_body(a0_ref, a1_ref, z_ref, d_ref, b_ref, o_ref):
    t = d_ref[...] * (a0_ref[...] + a1_ref[...] - z_ref[...]) + b_ref[...]
    o_ref[...] = jnp.where(t >= 0.0, 2.0 * t, 1.01 * t)


_BM = 400


def _mm(x, W, dinv_col):
    return pl.pallas_call(
        _mm_body,
        grid=(N // _BM,),
        in_specs=[
            pl.BlockSpec((_BM, CH), lambda i: (i, 0)),
            pl.BlockSpec((CH, CH), lambda i: (0, 0)),
            pl.BlockSpec((_BM, 1), lambda i: (i, 0)),
        ],
        out_specs=pl.BlockSpec((_BM, CH), lambda i: (i, 0)),
        out_shape=jax.ShapeDtypeStruct((N, CH), _f32),
    )(x, W, dinv_col)


def _fin(a0, a1, z, dinv_col, b2):
    return pl.pallas_call(
        _fin_body,
        grid=(N // _BM,),
        in_specs=[
            pl.BlockSpec((_BM, CH), lambda i: (i, 0)),
            pl.BlockSpec((_BM, CH), lambda i: (i, 0)),
            pl.BlockSpec((_BM, CH), lambda i: (i, 0)),
            pl.BlockSpec((_BM, 1), lambda i: (i, 0)),
            pl.BlockSpec((1, CH), lambda i: (0, 0)),
        ],
        out_specs=pl.BlockSpec((_BM, CH), lambda i: (i, 0)),
        out_shape=jax.ShapeDtypeStruct((N, CH), _f32),
    )(a0, a1, z, dinv_col, b2)


@jax.jit
def kernel(x, edges, W, b):
    src = edges[0].astype(jnp.int32).reshape(NW, KA, BA)
    dst = edges[1].astype(jnp.int32).reshape(NW, KA, BA)
    idx = jnp.stack([src, dst], axis=2)
    pad = jnp.broadcast_to(N + jnp.arange(EPAD, dtype=jnp.int32), (NW, EPAD))
    dst_deg = jnp.concatenate(
        [edges[1].astype(jnp.int32).reshape(NW, EPW), pad], axis=1
    ).reshape(NW, KD, BD)
    hist = _deg_kernel(dst_deg).reshape(2, NPAD // CH, CH)
    dinv = _rsq(hist[0], hist[1]).reshape(NPAD)[:N]
    dinv_col = dinv[:, None]
    z = _mm(x, W, dinv_col)
    acc = _agg_kernel(z, idx)
    return _fin(acc[0], acc[1], z, dinv_col, b[None, :])
